# baseline (speedup 1.0000x reference)
import functools

import jax
import jax.numpy as jnp
from jax import lax
from jax.experimental import pallas as pl
from jax.experimental.pallas import tpu as pltpu
from jax.experimental.pallas import tpu_sc as plsc

_BK = 4096
_L = 16
_NS = 16
_NEG = -1e30
_NEG_INIT = -3e38


def _tc_scores_body(n_rows, q_ref, k_ref, o_ref):
    i = pl.program_id(0)
    q = q_ref[...]
    qn = q / (jnp.sqrt(jnp.sum(q * q)) + 1e-8)
    kb = k_ref[...]
    dim = kb.shape[1]
    nchunk = dim // 128
    m_dot = kb[:, 0:128] * qn[:, 0:128]
    m_sq = kb[:, 0:128] * kb[:, 0:128]
    for a in range(1, nchunk):
        sl = slice(a * 128, (a + 1) * 128)
        m_dot = m_dot + kb[:, sl] * qn[:, sl]
        m_sq = m_sq + kb[:, sl] * kb[:, sl]
    c = jnp.concatenate([m_dot, m_sq], axis=1)
    io0 = lax.broadcasted_iota(jnp.int32, (256, 2), 0)
    io1 = lax.broadcasted_iota(jnp.int32, (256, 2), 1)
    e = ((io0 < 128) == (io1 == 0)).astype(jnp.float32)
    r = lax.dot_general(c, e, (((1,), (0,)), ((), ())),
                        preferred_element_type=jnp.float32,
                        precision=lax.Precision.HIGHEST)
    dot = r[:, 0:1]
    nsq = r[:, 1:2]
    s = dot / (jnp.sqrt(nsq) + 1e-8)
    rid = lax.broadcasted_iota(jnp.int32, s.shape, 0) + i * s.shape[0]
    o_ref[...] = jnp.where(rid < n_rows, s, _NEG)


def _tc_scores(q2, keys, total):
    n, dim = keys.shape
    grid = total // _BK
    return pl.pallas_call(
        functools.partial(_tc_scores_body, n),
        grid=(grid,),
        in_specs=[
            pl.BlockSpec((1, dim), lambda i: (0, 0)),
            pl.BlockSpec((_BK, dim), lambda i: (i, 0)),
        ],
        out_specs=pl.BlockSpec((_BK, 1), lambda i: (i, 0)),
        out_shape=jax.ShapeDtypeStruct((total, 1), jnp.float32),
        compiler_params=pltpu.CompilerParams(
            dimension_semantics=("arbitrary",)),
    )(q2, keys)


def _merge_top16(rv, ri, v, vi):
    sv, si = plsc.sort_key_val(v, vi)
    rsv = lax.rev(sv, (0,))
    rsi = lax.rev(si, (0,))
    take = (rsv > rv) | ((rsv == rv) & (rsi < ri))
    mv = jnp.where(take, rsv, rv)
    mi = jnp.where(take, rsi, ri)
    out = plsc.sort_key_val(mv, mi)
    return out[0], out[1]


def _make_sc_topk(total, dim):
    per_tile = total // _NS
    n_vregs = per_tile // _L
    mesh = plsc.VectorSubcoreMesh(core_axis_name="c", subcore_axis_name="s")

    @functools.partial(
        pl.kernel, mesh=mesh,
        out_type=jax.ShapeDtypeStruct((8, dim), jnp.float32),
        compiler_params=pltpu.CompilerParams(needs_layout_passes=False),
        scratch_types=[
            pltpu.VMEM((per_tile,), jnp.float32),
            pltpu.VMEM((_L,), jnp.float32),
            pltpu.VMEM((_L,), jnp.int32),
            pltpu.VMEM_SHARED((_NS * _L,), jnp.float32),
            pltpu.VMEM_SHARED((_NS * _L,), jnp.int32),
            pltpu.VMEM((_NS * _L,), jnp.float32),
            pltpu.VMEM((_NS * _L,), jnp.int32),
            pltpu.VMEM((_L,), jnp.int32),
            pltpu.VMEM((_L,), jnp.int32),
            pltpu.VMEM((_L, dim), jnp.float32),
            pltpu.SemaphoreType.DMA,
        ],
    )
    def sc_topk(scores_hbm, values_hbm, shift_hbm, out_hbm,
                sc_v, tv_v, ti_v, sh_v, sh_i, cv_v, ci_v,
                shf_v, gi_v, rows_v, sem):
        cid = lax.axis_index("c")
        sid = lax.axis_index("s")
        base = sid * per_tile
        pltpu.sync_copy(scores_hbm.at[pl.ds(base, per_tile)], sc_v)
        lane = lax.iota(jnp.int32, _L)
        rv0 = jnp.full((_L,), _NEG_INIT, jnp.float32)
        ri0 = jnp.full((_L,), 0x3FFFFFFF, jnp.int32)

        def body(j, carry):
            rv, ri, th = carry
            v = sc_v[pl.ds(j * _L, _L)]
            mx = jnp.max(v)

            def hit(_):
                vi = lane + (base + j * _L)
                nrv, nri = _merge_top16(rv, ri, v, vi)
                return nrv, nri, jnp.min(nrv)

            def miss(_):
                return rv, ri, th

            return lax.cond(mx >= th, hit, miss, None)

        rv, ri, _ = lax.fori_loop(
            0, n_vregs, body, (rv0, ri0, jnp.float32(_NEG_INIT)))
        tv_v[...] = rv
        ti_v[...] = ri
        pltpu.sync_copy(tv_v, sh_v.at[pl.ds(sid * _L, _L)])
        pltpu.sync_copy(ti_v, sh_i.at[pl.ds(sid * _L, _L)])
        plsc.subcore_barrier()

        @pl.when((sid == 0) & (cid == 0))
        def _():
            pltpu.sync_copy(sh_v, cv_v)
            pltpu.sync_copy(sh_i, ci_v)
            pltpu.sync_copy(shift_hbm, shf_v)
            fv = jnp.full((_L,), _NEG_INIT, jnp.float32)
            fi = jnp.full((_L,), 0x3FFFFFFF, jnp.int32)
            for t in range(_NS):
                cv = cv_v[pl.ds(t * _L, _L)]
                ci = ci_v[pl.ds(t * _L, _L)]
                fv, fi = _merge_top16(fv, fi, cv, ci)
            gi_v[...] = lax.rev(fi, (0,)) + shf_v[...]
            pltpu.async_copy(values_hbm.at[gi_v], rows_v, sem).wait()
            pltpu.sync_copy(rows_v.at[pl.ds(0, 8)], out_hbm)

    return sc_topk


def kernel(query, keys, values, top_k):
    n, dim = keys.shape
    total = ((n + _BK - 1) // _BK) * _BK
    q2 = query.reshape(1, dim).astype(jnp.float32)
    scores = _tc_scores(q2, keys, total)
    shift = jnp.full((_L,), jnp.asarray(top_k, jnp.int32) - 8, jnp.int32)
    sc_topk = _make_sc_topk(total, dim)
    return sc_topk(scores.reshape(total), values, shift)


# speedup vs baseline: 1.0522x; 1.0522x over previous
import functools

import jax
import jax.numpy as jnp
from jax import lax
from jax.experimental import pallas as pl
from jax.experimental.pallas import tpu as pltpu
from jax.experimental.pallas import tpu_sc as plsc

_BK = 4096
_L = 16
_NS = 16
_NEG = -1e30
_NEG_INIT = -3e38


def _tc_scores_body(n_rows, q_ref, kl_ref, kr_ref, o_ref):
    i = pl.program_id(0)
    q = q_ref[...]
    qn = q / (jnp.sqrt(jnp.sum(q * q)) + 1e-8)
    dim = q.shape[1]
    half = dim // 2
    m_dot = None
    m_sq = None
    for h, kb in ((0, kl_ref[...]), (1, kr_ref[...])):
        for a in range(half // 128):
            sl = slice(a * 128, (a + 1) * 128)
            qsl = slice(h * half + a * 128, h * half + (a + 1) * 128)
            d = kb[:, sl] * qn[:, qsl]
            s2 = kb[:, sl] * kb[:, sl]
            m_dot = d if m_dot is None else m_dot + d
            m_sq = s2 if m_sq is None else m_sq + s2
    c = jnp.concatenate([m_dot, m_sq], axis=1)
    io0 = lax.broadcasted_iota(jnp.int32, (256, 2), 0)
    io1 = lax.broadcasted_iota(jnp.int32, (256, 2), 1)
    e = ((io0 < 128) == (io1 == 0)).astype(jnp.float32)
    r = lax.dot_general(c, e, (((1,), (0,)), ((), ())),
                        preferred_element_type=jnp.float32,
                        precision=lax.Precision.HIGHEST)
    dot = r[:, 0:1]
    nsq = r[:, 1:2]
    s = dot / (jnp.sqrt(nsq) + 1e-8)
    rid = lax.broadcasted_iota(jnp.int32, s.shape, 0) + i * s.shape[0]
    o_ref[...] = jnp.where(rid < n_rows, s, _NEG)


def _tc_scores(q2, keys, total):
    n, dim = keys.shape
    grid = total // _BK
    return pl.pallas_call(
        functools.partial(_tc_scores_body, n),
        grid=(grid,),
        in_specs=[
            pl.BlockSpec((1, dim), lambda i: (0, 0)),
            pl.BlockSpec((_BK, dim // 2), lambda i: (i, 0)),
            pl.BlockSpec((_BK, dim // 2), lambda i: (i, 1)),
        ],
        out_specs=pl.BlockSpec((_BK, 1), lambda i: (i, 0)),
        out_shape=jax.ShapeDtypeStruct((total, 1), jnp.float32),
        compiler_params=pltpu.CompilerParams(
            dimension_semantics=("arbitrary",)),
    )(q2, keys, keys)


def _merge_top16(rv, ri, v, vi):
    sv, si = plsc.sort_key_val(v, vi)
    rsv = lax.rev(sv, (0,))
    rsi = lax.rev(si, (0,))
    take = (rsv > rv) | ((rsv == rv) & (rsi < ri))
    mv = jnp.where(take, rsv, rv)
    mi = jnp.where(take, rsi, ri)
    out = plsc.sort_key_val(mv, mi)
    return out[0], out[1]


def _make_sc_topk(total, dim):
    per_tile = total // _NS
    n_vregs = per_tile // _L
    mesh = plsc.VectorSubcoreMesh(core_axis_name="c", subcore_axis_name="s")

    @functools.partial(
        pl.kernel, mesh=mesh,
        out_type=jax.ShapeDtypeStruct((8, dim), jnp.float32),
        compiler_params=pltpu.CompilerParams(needs_layout_passes=False),
        scratch_types=[
            pltpu.VMEM((per_tile,), jnp.float32),
            pltpu.VMEM((_L,), jnp.float32),
            pltpu.VMEM((_L,), jnp.int32),
            pltpu.VMEM_SHARED((_NS * _L,), jnp.float32),
            pltpu.VMEM_SHARED((_NS * _L,), jnp.int32),
            pltpu.VMEM((_NS * _L,), jnp.float32),
            pltpu.VMEM((_NS * _L,), jnp.int32),
            pltpu.VMEM((_L,), jnp.int32),
            pltpu.VMEM((_L,), jnp.int32),
            pltpu.VMEM((_L, dim), jnp.float32),
            pltpu.SemaphoreType.DMA,
        ],
    )
    def sc_topk(scores_hbm, values_hbm, shift_hbm, out_hbm,
                sc_v, tv_v, ti_v, sh_v, sh_i, cv_v, ci_v,
                shf_v, gi_v, rows_v, sem):
        cid = lax.axis_index("c")
        sid = lax.axis_index("s")
        base = sid * per_tile
        pltpu.sync_copy(scores_hbm.at[pl.ds(base, per_tile)], sc_v)
        lane = lax.iota(jnp.int32, _L)
        rv0 = jnp.full((_L,), _NEG_INIT, jnp.float32)
        ri0 = jnp.full((_L,), 0x3FFFFFFF, jnp.int32)

        def body(j, carry):
            rv, ri = carry
            v = sc_v[pl.ds(j * _L, _L)]
            vi = lane + (base + j * _L)
            return _merge_top16(rv, ri, v, vi)

        rv, ri = lax.fori_loop(0, n_vregs, body, (rv0, ri0))
        tv_v[...] = rv
        ti_v[...] = ri
        pltpu.sync_copy(tv_v, sh_v.at[pl.ds(sid * _L, _L)])
        pltpu.sync_copy(ti_v, sh_i.at[pl.ds(sid * _L, _L)])
        plsc.subcore_barrier()

        @pl.when((sid == 0) & (cid == 0))
        def _():
            pltpu.sync_copy(sh_v, cv_v)
            pltpu.sync_copy(sh_i, ci_v)
            pltpu.sync_copy(shift_hbm, shf_v)
            fv = jnp.full((_L,), _NEG_INIT, jnp.float32)
            fi = jnp.full((_L,), 0x3FFFFFFF, jnp.int32)
            for t in range(_NS):
                cv = cv_v[pl.ds(t * _L, _L)]
                ci = ci_v[pl.ds(t * _L, _L)]
                fv, fi = _merge_top16(fv, fi, cv, ci)
            gi_v[...] = lax.rev(fi, (0,)) + shf_v[...]
            pltpu.async_copy(values_hbm.at[gi_v], rows_v, sem).wait()
            pltpu.sync_copy(rows_v.at[pl.ds(0, 8)], out_hbm)

    return sc_topk


def kernel(query, keys, values, top_k):
    n, dim = keys.shape
    total = ((n + _BK - 1) // _BK) * _BK
    q2 = query.reshape(1, dim).astype(jnp.float32)
    scores = _tc_scores(q2, keys, total)
    shift = jnp.full((_L,), jnp.asarray(top_k, jnp.int32) - 8, jnp.int32)
    sc_topk = _make_sc_topk(total, dim)
    return sc_topk(scores.reshape(total), values, shift)


# speedup vs baseline: 1.0848x; 1.0310x over previous
import functools

import jax
import jax.numpy as jnp
from jax import lax
from jax.experimental import pallas as pl
from jax.experimental.pallas import tpu as pltpu
from jax.experimental.pallas import tpu_sc as plsc

_BK = 4096
_L = 16
_NS = 16
_NEG = -1e30
_NEG_INIT = -3e38


def _tc_scores_body(n_rows, q_ref, k_ref, o_ref):
    i = pl.program_id(0)
    q = q_ref[...]
    qn = q / (jnp.sqrt(jnp.sum(q * q)) + 1e-8)
    kb = k_ref[...]
    dim = kb.shape[1]
    nchunk = dim // 128
    m_dot = kb[:, 0:128] * qn[:, 0:128]
    m_sq = kb[:, 0:128] * kb[:, 0:128]
    for a in range(1, nchunk):
        sl = slice(a * 128, (a + 1) * 128)
        m_dot = m_dot + kb[:, sl] * qn[:, sl]
        m_sq = m_sq + kb[:, sl] * kb[:, sl]
    c = jnp.concatenate([m_dot, m_sq], axis=1)
    io0 = lax.broadcasted_iota(jnp.int32, (256, 2), 0)
    io1 = lax.broadcasted_iota(jnp.int32, (256, 2), 1)
    e = ((io0 < 128) == (io1 == 0)).astype(jnp.float32)
    r = lax.dot_general(c, e, (((1,), (0,)), ((), ())),
                        preferred_element_type=jnp.float32,
                        precision=lax.Precision.HIGHEST)
    dot = r[:, 0:1]
    nsq = r[:, 1:2]
    s = dot / (jnp.sqrt(nsq) + 1e-8)
    rid = lax.broadcasted_iota(jnp.int32, s.shape, 0) + i * s.shape[0]
    o_ref[...] = jnp.where(rid < n_rows, s, _NEG)


def _tc_scores(q2, keys, total):
    n, dim = keys.shape
    grid = total // _BK
    return pl.pallas_call(
        functools.partial(_tc_scores_body, n),
        grid=(grid,),
        in_specs=[
            pl.BlockSpec((1, dim), lambda i: (0, 0)),
            pl.BlockSpec((_BK, dim), lambda i: (i, 0)),
        ],
        out_specs=pl.BlockSpec((_BK, 1), lambda i: (i, 0)),
        out_shape=jax.ShapeDtypeStruct((total, 1), jnp.float32),
        compiler_params=pltpu.CompilerParams(
            dimension_semantics=("arbitrary",)),
    )(q2, keys)


def _merge_top16(rv, ri, v, vi):
    sv, si = plsc.sort_key_val(v, vi)
    rsv = lax.rev(sv, (0,))
    rsi = lax.rev(si, (0,))
    take = (rsv > rv) | ((rsv == rv) & (rsi < ri))
    mv = jnp.where(take, rsv, rv)
    mi = jnp.where(take, rsi, ri)
    out = plsc.sort_key_val(mv, mi)
    return out[0], out[1]


def _make_sc_topk(total, dim):
    per_tile = total // _NS
    n_vregs = per_tile // _L
    mesh = plsc.VectorSubcoreMesh(core_axis_name="c", subcore_axis_name="s")

    @functools.partial(
        pl.kernel, mesh=mesh,
        out_type=jax.ShapeDtypeStruct((8, dim), jnp.float32),
        compiler_params=pltpu.CompilerParams(needs_layout_passes=False),
        scratch_types=[
            pltpu.VMEM((per_tile,), jnp.float32),
            pltpu.VMEM((_L,), jnp.float32),
            pltpu.VMEM((_L,), jnp.int32),
            pltpu.VMEM_SHARED((_NS * _L,), jnp.float32),
            pltpu.VMEM_SHARED((_NS * _L,), jnp.int32),
            pltpu.VMEM((_NS * _L,), jnp.float32),
            pltpu.VMEM((_NS * _L,), jnp.int32),
            pltpu.VMEM((_L,), jnp.int32),
            pltpu.VMEM((_L,), jnp.int32),
            pltpu.VMEM((_L, dim), jnp.float32),
            pltpu.SemaphoreType.DMA,
        ],
    )
    def sc_topk(scores_hbm, values_hbm, shift_hbm, out_hbm,
                sc_v, tv_v, ti_v, sh_v, sh_i, cv_v, ci_v,
                shf_v, gi_v, rows_v, sem):
        cid = lax.axis_index("c")
        sid = lax.axis_index("s")
        base = sid * per_tile
        pltpu.sync_copy(scores_hbm.at[pl.ds(base, per_tile)], sc_v)
        lane = lax.iota(jnp.int32, _L)
        rv0 = jnp.full((_L,), _NEG_INIT, jnp.float32)
        ri0 = jnp.full((_L,), 0x3FFFFFFF, jnp.int32)

        nch = 4

        def body(j, carry):
            out = []
            for k in range(nch):
                rv, ri = carry[2 * k], carry[2 * k + 1]
                off = (j * nch + k) * _L
                v = sc_v[pl.ds(off, _L)]
                vi = lane + (base + off)
                nrv, nri = _merge_top16(rv, ri, v, vi)
                out.extend((nrv, nri))
            return tuple(out)

        carry = lax.fori_loop(0, n_vregs // nch, body, (rv0, ri0) * nch)
        rv, ri = carry[0], carry[1]
        for k in range(1, nch):
            rv, ri = _merge_top16(rv, ri, carry[2 * k], carry[2 * k + 1])
        tv_v[...] = rv
        ti_v[...] = ri
        pltpu.sync_copy(tv_v, sh_v.at[pl.ds(sid * _L, _L)])
        pltpu.sync_copy(ti_v, sh_i.at[pl.ds(sid * _L, _L)])
        plsc.subcore_barrier()

        @pl.when((sid == 0) & (cid == 0))
        def _():
            pltpu.sync_copy(sh_v, cv_v)
            pltpu.sync_copy(sh_i, ci_v)
            pltpu.sync_copy(shift_hbm, shf_v)
            fv = jnp.full((_L,), _NEG_INIT, jnp.float32)
            fi = jnp.full((_L,), 0x3FFFFFFF, jnp.int32)
            for t in range(_NS):
                cv = cv_v[pl.ds(t * _L, _L)]
                ci = ci_v[pl.ds(t * _L, _L)]
                fv, fi = _merge_top16(fv, fi, cv, ci)
            gi_v[...] = lax.rev(fi, (0,)) + shf_v[...]
            pltpu.async_copy(values_hbm.at[gi_v], rows_v, sem).wait()
            pltpu.sync_copy(rows_v.at[pl.ds(0, 8)], out_hbm)

    return sc_topk


def kernel(query, keys, values, top_k):
    n, dim = keys.shape
    total = ((n + _BK - 1) // _BK) * _BK
    q2 = query.reshape(1, dim).astype(jnp.float32)
    scores = _tc_scores(q2, keys, total)
    shift = jnp.full((_L,), jnp.asarray(top_k, jnp.int32) - 8, jnp.int32)
    sc_topk = _make_sc_topk(total, dim)
    return sc_topk(scores.reshape(total), values, shift)
